# baseline (speedup 1.0000x reference)
import jax
import jax.numpy as jnp
from jax.experimental import pallas as pl
from jax.experimental.pallas import tpu as pltpu

_INTERPRET = False

B, L, D = 2, 1024, 1024
DI = 2048
NS = 16
DTR = 64
KC = 4
EPS = 1e-5

TDI = 128
NB = DI // TDI
LT = 256
NLT = L // LT


def _silu(v):
    return v * jax.nn.sigmoid(v)


def _proj_kernel(x_ref, nw_ref, win_ref, cw_ref, cb_ref, o_ref, carry):
    j = pl.program_id(1)
    lt = pl.program_id(2)
    x = x_ref[0]
    ssq = jnp.mean(x * x, axis=-1, keepdims=True)
    xn = x * jax.lax.rsqrt(ssq + EPS) * nw_ref[0]
    xz = jnp.dot(xn, win_ref[0], preferred_element_type=jnp.float32)

    @pl.when(lt == 0)
    def _():
        carry[...] = jnp.zeros((KC - 1, DI), jnp.float32)

    w = cw_ref[0]
    xfull = jnp.concatenate([carry[...], xz], axis=0)
    carry[...] = xz[LT - (KC - 1):, :]
    acc = cb_ref[0]
    for k in range(KC):
        acc = acc + xfull[k:k + LT] * w[k][None, :]
    o_ref[0, 0] = jnp.where(j == 0, _silu(acc), xz)


def _proj(l, x, nw, win, cwT, cb):
    return pl.pallas_call(
        _proj_kernel,
        out_shape=jax.ShapeDtypeStruct((B, 2, L, DI), jnp.float32),
        grid=(B, 2, NLT),
        in_specs=[
            pl.BlockSpec((1, LT, D), lambda b, j, lt: (b, lt, 0)),
            pl.BlockSpec((1, 1, D), lambda b, j, lt: (l, 0, 0)),
            pl.BlockSpec((1, D, DI), lambda b, j, lt: (l, 0, j)),
            pl.BlockSpec((1, KC, DI), lambda b, j, lt: (l, 0, 0)),
            pl.BlockSpec((1, 1, DI), lambda b, j, lt: (l, 0, 0)),
        ],
        out_specs=pl.BlockSpec((1, 1, LT, DI), lambda b, j, lt: (b, j, lt, 0)),
        scratch_shapes=[pltpu.VMEM((KC - 1, DI), jnp.float32)],
        compiler_params=pltpu.CompilerParams(
            dimension_semantics=("parallel", "arbitrary", "arbitrary"),
            vmem_limit_bytes=48 * 1024 * 1024,
        ),
        name="mamba_proj",
        interpret=_INTERPRET,
    )(x, nw, win, cwT, cb)


_MT = 256
_NMT = B * L // _MT


def _dbc_kernel(xc_ref, wx_ref, wdt_ref, bdt_ref, pb_ref, d_ref, b_ref, c_ref):
    xc = xc_ref[0, 0]
    dbc = jnp.dot(xc, wx_ref[0], preferred_element_type=jnp.float32)
    dt = dbc[:, :DTR]
    d_ref[0] = jax.nn.softplus(
        jnp.dot(dt, wdt_ref[0], preferred_element_type=jnp.float32)
        + bdt_ref[0])
    pb = pb_ref[...]
    b_ref[0] = jnp.dot(dbc[:, DTR:DTR + NS], pb,
                       preferred_element_type=jnp.float32)
    c_ref[0] = jnp.dot(dbc[:, DTR + NS:DTR + 2 * NS], pb,
                       preferred_element_type=jnp.float32)


def _dbc(l, o, wx, wdt, bdt2, pb):
    nlt = L // _MT
    return pl.pallas_call(
        _dbc_kernel,
        out_shape=(
            jax.ShapeDtypeStruct((B, L, DI), jnp.float32),
            jax.ShapeDtypeStruct((B, L, NS * 128), jnp.float32),
            jax.ShapeDtypeStruct((B, L, NS * 128), jnp.float32),
        ),
        grid=(_NMT,),
        in_specs=[
            pl.BlockSpec((1, 1, _MT, DI), lambda i: (i // nlt, 0, i % nlt, 0)),
            pl.BlockSpec((1, DI, DTR + 2 * NS), lambda i: (l, 0, 0)),
            pl.BlockSpec((1, DTR, DI), lambda i: (l, 0, 0)),
            pl.BlockSpec((1, 1, DI), lambda i: (l, 0, 0)),
            pl.BlockSpec((NS, NS * 128), lambda i: (0, 0)),
        ],
        out_specs=(
            pl.BlockSpec((1, _MT, DI), lambda i: (i // nlt, i % nlt, 0)),
            pl.BlockSpec((1, _MT, NS * 128), lambda i: (i // nlt, i % nlt, 0)),
            pl.BlockSpec((1, _MT, NS * 128), lambda i: (i // nlt, i % nlt, 0)),
        ),
        compiler_params=pltpu.CompilerParams(
            dimension_semantics=("parallel",),
            vmem_limit_bytes=48 * 1024 * 1024,
        ),
        name="mamba_dbc",
        interpret=_INTERPRET,
    )(o, wx, wdt, bdt2, pb)


_PC = 128
L2 = L // 2


def _scan_kernel(u_ref, d_ref, bm_ref, cm_ref, z_ref, at_ref, dp_ref,
                 o_ref, dA_s, dBu_s):
    A = -jnp.exp(at_ref[0])

    def pre(i, _):
        sl = pl.ds(i * _PC, _PC)
        delta = d_ref[0, sl, :]
        u = u_ref[0, 0, sl, :]
        dA = jnp.exp(delta[:, None, :] * A[None, :, :])
        bm4 = bm_ref[0, sl, :, :]
        dBu = (delta * u)[:, None, :] * bm4
        a4 = dA.reshape(_PC // 2, 2, NS, TDI)
        b4 = dBu.reshape(_PC // 2, 2, NS, TDI)
        a0 = a4[:, 0]
        a1 = a4[:, 1]
        b0 = b4[:, 0]
        b1 = b4[:, 1]
        sl2 = pl.ds(i * (_PC // 2), _PC // 2)
        dA_s[sl2, 0] = a0
        dA_s[sl2, 1] = a1 * a0
        dBu_s[sl2, 0] = b0
        dBu_s[sl2, 1] = a1 * b0 + b1
        return 0

    jax.lax.fori_loop(0, L // _PC, pre, 0)

    def step(k, h):
        cA = dA_s[k, 1]
        cB = dBu_s[k, 1]
        dBu_s[k, 1] = h
        h = cA * h + cB
        dA_s[k, 1] = h
        return h

    jax.lax.fori_loop(0, L2, step, jnp.zeros((NS, TDI), jnp.float32),
                      unroll=4)

    dp = dp_ref[0]
    _C2 = 64

    def post(i, _):
        sl2 = pl.ds(i * _C2, _C2)
        sl = pl.ds(i * 2 * _C2, 2 * _C2)
        h_odd = dA_s[sl2, 1]
        hm1 = dBu_s[sl2, 1]
        h_even = dA_s[sl2, 0] * hm1 + dBu_s[sl2, 0]
        hist4 = jnp.stack([h_even, h_odd], axis=1)
        cm4 = cm_ref[0, sl, :, :].reshape(_C2, 2, NS, TDI)
        y = jnp.sum(hist4 * cm4, axis=2)
        y = y.reshape(2 * _C2, TDI)
        y = y + u_ref[0, 0, sl, :] * dp
        zv = z_ref[0, 0, sl, :]
        o_ref[0, sl, :] = y * _silu(zv)
        return 0

    jax.lax.fori_loop(0, L2 // _C2, post, 0)


def _scan(l, o, delta, bm4, cm4, a_logT, dp2):
    grid = (B * NB,)
    return pl.pallas_call(
        _scan_kernel,
        out_shape=jax.ShapeDtypeStruct((B, L, DI), jnp.float32),
        grid=grid,
        in_specs=[
            pl.BlockSpec((1, 1, L, TDI), lambda i: (i // NB, 0, 0, i % NB)),
            pl.BlockSpec((1, L, TDI), lambda i: (i // NB, 0, i % NB)),
            pl.BlockSpec((1, L, NS, TDI), lambda i: (i // NB, 0, 0, 0)),
            pl.BlockSpec((1, L, NS, TDI), lambda i: (i // NB, 0, 0, 0)),
            pl.BlockSpec((1, 1, L, TDI), lambda i: (i // NB, 1, 0, i % NB)),
            pl.BlockSpec((1, NS, TDI), lambda i: (l, 0, i % NB)),
            pl.BlockSpec((1, 1, TDI), lambda i: (l, 0, i % NB)),
        ],
        out_specs=pl.BlockSpec((1, L, TDI), lambda i: (i // NB, 0, i % NB)),
        scratch_shapes=[
            pltpu.VMEM((L2, 2, NS, TDI), jnp.float32),
            pltpu.VMEM((L2, 2, NS, TDI), jnp.float32),
        ],
        compiler_params=pltpu.CompilerParams(
            dimension_semantics=("parallel",),
            vmem_limit_bytes=55 * 1024 * 1024,
        ),
        name="mamba_scan",
        interpret=_INTERPRET,
    )(o, delta, bm4, cm4, o, a_logT, dp2)


def _down_kernel(yz_ref, w_ref, x_ref, o_ref):
    o_ref[0] = x_ref[0] + jnp.dot(
        yz_ref[0], w_ref[0], preferred_element_type=jnp.float32)


def _down(l, yz, wblk, x):
    nlt = L // _MT
    return pl.pallas_call(
        _down_kernel,
        out_shape=jax.ShapeDtypeStruct((B, L, D), jnp.float32),
        grid=(_NMT,),
        in_specs=[
            pl.BlockSpec((1, _MT, DI), lambda i: (i // nlt, i % nlt, 0)),
            pl.BlockSpec((1, DI, D), lambda i: (l, 0, 0)),
            pl.BlockSpec((1, _MT, D), lambda i: (i // nlt, i % nlt, 0)),
        ],
        out_specs=pl.BlockSpec((1, _MT, D), lambda i: (i // nlt, i % nlt, 0)),
        compiler_params=pltpu.CompilerParams(
            dimension_semantics=("parallel",),
            vmem_limit_bytes=48 * 1024 * 1024,
        ),
        name="mamba_down",
        interpret=_INTERPRET,
    )(yz, wblk, x)


def _out_kernel(h_ref, w_ref, b_ref, o_ref):
    o_ref[...] = jnp.dot(
        h_ref[...], w_ref[...], preferred_element_type=jnp.float32) + b_ref[...]


def _out_proj(h2, wout, bout2):
    M = B * L
    OD = wout.shape[1]
    return pl.pallas_call(
        _out_kernel,
        out_shape=jax.ShapeDtypeStruct((M, OD), jnp.float32),
        grid=(M // _MT,),
        in_specs=[
            pl.BlockSpec((_MT, D), lambda i: (i, 0)),
            pl.BlockSpec((D, OD), lambda i: (0, 0)),
            pl.BlockSpec((1, OD), lambda i: (0, 0)),
        ],
        out_specs=pl.BlockSpec((_MT, OD), lambda i: (i, 0)),
        compiler_params=pltpu.CompilerParams(
            dimension_semantics=("parallel",),
            vmem_limit_bytes=48 * 1024 * 1024,
        ),
        name="mamba_out",
        interpret=_INTERPRET,
    )(h2, wout, bout2)


def kernel(x, norm_w, Win, conv_w, conv_b, Wx, Wdt, bdt, A_log, Dp, Wblk,
           Wout, bout):
    h = x
    nw = norm_w[:, None, :]
    cwT = conv_w.transpose(0, 2, 1)
    cb = conv_b[:, None, :]
    bdt2 = bdt[:, None, :]
    a_logT = A_log.transpose(0, 2, 1)
    dp2 = Dp[:, None, :]
    pb = jnp.kron(jnp.eye(NS, dtype=jnp.float32),
                  jnp.ones((1, 128), jnp.float32))
    for l in range(Win.shape[0]):
        o = _proj(l, h, nw, Win, cwT, cb)
        delta, bmf, cmf = _dbc(l, o, Wx, Wdt, bdt2, pb)
        bm4 = bmf.reshape(B, L, NS, 128)
        cm4 = cmf.reshape(B, L, NS, 128)
        yz = _scan(l, o, delta, bm4, cm4, a_logT, dp2)
        h = _down(l, yz, Wblk, h)

    out2 = _out_proj(h.reshape(B * L, D), Wout, bout[None, :])
    return out2.reshape(B, L, Wout.shape[1])


# speedup vs baseline: 1.2441x; 1.2441x over previous
import jax
import jax.numpy as jnp
from jax.experimental import pallas as pl
from jax.experimental.pallas import tpu as pltpu

_INTERPRET = False

B, L, D = 2, 1024, 1024
DI = 2048
NS = 16
DTR = 64
KC = 4
EPS = 1e-5

TDI = 256
NB = DI // TDI
LT = 256
NLT = L // LT


def _silu(v):
    return v * jax.nn.sigmoid(v)


def _proj_kernel(x_ref, nw_ref, win_ref, cw_ref, cb_ref, o_ref, carry):
    j = pl.program_id(1)
    lt = pl.program_id(2)
    x = x_ref[0]
    ssq = jnp.mean(x * x, axis=-1, keepdims=True)
    xn = x * jax.lax.rsqrt(ssq + EPS) * nw_ref[0]
    xz = jnp.dot(xn, win_ref[0], preferred_element_type=jnp.float32)

    @pl.when(lt == 0)
    def _():
        carry[...] = jnp.zeros((KC - 1, DI), jnp.float32)

    w = cw_ref[0]
    xfull = jnp.concatenate([carry[...], xz], axis=0)
    carry[...] = xz[LT - (KC - 1):, :]
    acc = cb_ref[0]
    for k in range(KC):
        acc = acc + xfull[k:k + LT] * w[k][None, :]
    o_ref[0, 0] = jnp.where(j == 0, _silu(acc), xz)


def _proj(l, x, nw, win, cwT, cb):
    return pl.pallas_call(
        _proj_kernel,
        out_shape=jax.ShapeDtypeStruct((B, 2, L, DI), jnp.float32),
        grid=(B, 2, NLT),
        in_specs=[
            pl.BlockSpec((1, LT, D), lambda b, j, lt: (b, lt, 0)),
            pl.BlockSpec((1, 1, D), lambda b, j, lt: (l, 0, 0)),
            pl.BlockSpec((1, D, DI), lambda b, j, lt: (l, 0, j)),
            pl.BlockSpec((1, KC, DI), lambda b, j, lt: (l, 0, 0)),
            pl.BlockSpec((1, 1, DI), lambda b, j, lt: (l, 0, 0)),
        ],
        out_specs=pl.BlockSpec((1, 1, LT, DI), lambda b, j, lt: (b, j, lt, 0)),
        scratch_shapes=[pltpu.VMEM((KC - 1, DI), jnp.float32)],
        compiler_params=pltpu.CompilerParams(
            dimension_semantics=("parallel", "arbitrary", "arbitrary"),
            vmem_limit_bytes=48 * 1024 * 1024,
        ),
        name="mamba_proj",
        interpret=_INTERPRET,
    )(x, nw, win, cwT, cb)


_MT = 256
_NMT = B * L // _MT


def _dbc_kernel(xc_ref, wx_ref, wdt_ref, bdt_ref, pb_ref, d_ref, b_ref, c_ref):
    xc = xc_ref[0, 0]
    dbc = jnp.dot(xc, wx_ref[0], preferred_element_type=jnp.float32)
    dt = dbc[:, :DTR]
    d_ref[0] = jax.nn.softplus(
        jnp.dot(dt, wdt_ref[0], preferred_element_type=jnp.float32)
        + bdt_ref[0])
    pb = pb_ref[...]
    b_ref[0] = jnp.dot(dbc[:, DTR:DTR + NS], pb,
                       preferred_element_type=jnp.float32).astype(jnp.bfloat16)
    c_ref[0] = jnp.dot(dbc[:, DTR + NS:DTR + 2 * NS], pb,
                       preferred_element_type=jnp.float32).astype(jnp.bfloat16)


def _dbc(l, o, wx, wdt, bdt2, pb):
    nlt = L // _MT
    return pl.pallas_call(
        _dbc_kernel,
        out_shape=(
            jax.ShapeDtypeStruct((B, L, DI), jnp.float32),
            jax.ShapeDtypeStruct((B, L, NS * 128), jnp.bfloat16),
            jax.ShapeDtypeStruct((B, L, NS * 128), jnp.bfloat16),
        ),
        grid=(_NMT,),
        in_specs=[
            pl.BlockSpec((1, 1, _MT, DI), lambda i: (i // nlt, 0, i % nlt, 0)),
            pl.BlockSpec((1, DI, DTR + 2 * NS), lambda i: (l, 0, 0)),
            pl.BlockSpec((1, DTR, DI), lambda i: (l, 0, 0)),
            pl.BlockSpec((1, 1, DI), lambda i: (l, 0, 0)),
            pl.BlockSpec((NS, NS * 128), lambda i: (0, 0)),
        ],
        out_specs=(
            pl.BlockSpec((1, _MT, DI), lambda i: (i // nlt, i % nlt, 0)),
            pl.BlockSpec((1, _MT, NS * 128), lambda i: (i // nlt, i % nlt, 0)),
            pl.BlockSpec((1, _MT, NS * 128), lambda i: (i // nlt, i % nlt, 0)),
        ),
        compiler_params=pltpu.CompilerParams(
            dimension_semantics=("parallel",),
            vmem_limit_bytes=48 * 1024 * 1024,
        ),
        name="mamba_dbc",
        interpret=_INTERPRET,
    )(o, wx, wdt, bdt2, pb)


_PC = 128
L2 = L // 2


_RG = 8


def _scan_kernel(u_ref, d_ref, bm_ref, cm_ref, z_ref, at_ref, dp_ref,
                 o_ref, hist_s):
    A = -jnp.exp(at_ref[0])
    rep = TDI // 128

    def group(g, h):
        sl = pl.ds(g * _RG, _RG)
        d8 = d_ref[0, sl, :]
        u8 = u_ref[0, 0, sl, :]
        du8 = d8 * u8
        bm8 = bm_ref[0, sl, :, :].astype(jnp.float32)
        base = g * _RG
        for i in range(_RG):
            dA = jnp.exp(jnp.broadcast_to(d8[i:i + 1, :], (NS, TDI)) * A)
            bmt = pltpu.repeat(bm8[i], rep, axis=1)
            dBu = jnp.broadcast_to(du8[i:i + 1, :], (NS, TDI)) * bmt
            h = dA * h + dBu
            hist_s[base + i] = h
        return h

    jax.lax.fori_loop(0, L // _RG, group,
                      jnp.zeros((NS, TDI), jnp.float32))

    dp = dp_ref[0]
    _C = 128

    def post(i, _):
        sl = pl.ds(i * _C, _C)
        hist = hist_s[sl]
        cm4 = cm_ref[0, sl, :, :].astype(jnp.float32)
        cmr = pltpu.repeat(cm4.reshape(_C * NS, 128), rep,
                           axis=1).reshape(_C, NS, TDI)
        y = jnp.sum(hist * cmr, axis=1)
        y = y + u_ref[0, 0, sl, :] * dp
        zv = z_ref[0, 0, sl, :]
        o_ref[0, sl, :] = y * _silu(zv)
        return 0

    jax.lax.fori_loop(0, L // _C, post, 0)


def _scan(l, o, delta, bm4, cm4, a_logT, dp2):
    grid = (B * NB,)
    return pl.pallas_call(
        _scan_kernel,
        out_shape=jax.ShapeDtypeStruct((B, L, DI), jnp.float32),
        grid=grid,
        in_specs=[
            pl.BlockSpec((1, 1, L, TDI), lambda i: (i // NB, 0, 0, i % NB)),
            pl.BlockSpec((1, L, TDI), lambda i: (i // NB, 0, i % NB)),
            pl.BlockSpec((1, L, NS, 128), lambda i: (i // NB, 0, 0, 0)),
            pl.BlockSpec((1, L, NS, 128), lambda i: (i // NB, 0, 0, 0)),
            pl.BlockSpec((1, 1, L, TDI), lambda i: (i // NB, 1, 0, i % NB)),
            pl.BlockSpec((1, NS, TDI), lambda i: (l, 0, i % NB)),
            pl.BlockSpec((1, 1, TDI), lambda i: (l, 0, i % NB)),
        ],
        out_specs=pl.BlockSpec((1, L, TDI), lambda i: (i // NB, 0, i % NB)),
        scratch_shapes=[
            pltpu.VMEM((L, NS, TDI), jnp.float32),
        ],
        compiler_params=pltpu.CompilerParams(
            dimension_semantics=("parallel",),
            vmem_limit_bytes=55 * 1024 * 1024,
        ),
        name="mamba_scan",
        interpret=_INTERPRET,
    )(o, delta, bm4, cm4, o, a_logT, dp2)


def _down_kernel(yz_ref, w_ref, x_ref, o_ref):
    o_ref[0] = x_ref[0] + jnp.dot(
        yz_ref[0], w_ref[0], preferred_element_type=jnp.float32)


def _down(l, yz, wblk, x):
    nlt = L // _MT
    return pl.pallas_call(
        _down_kernel,
        out_shape=jax.ShapeDtypeStruct((B, L, D), jnp.float32),
        grid=(_NMT,),
        in_specs=[
            pl.BlockSpec((1, _MT, DI), lambda i: (i // nlt, i % nlt, 0)),
            pl.BlockSpec((1, DI, D), lambda i: (l, 0, 0)),
            pl.BlockSpec((1, _MT, D), lambda i: (i // nlt, i % nlt, 0)),
        ],
        out_specs=pl.BlockSpec((1, _MT, D), lambda i: (i // nlt, i % nlt, 0)),
        compiler_params=pltpu.CompilerParams(
            dimension_semantics=("parallel",),
            vmem_limit_bytes=48 * 1024 * 1024,
        ),
        name="mamba_down",
        interpret=_INTERPRET,
    )(yz, wblk, x)


def _out_kernel(h_ref, w_ref, b_ref, o_ref):
    o_ref[...] = jnp.dot(
        h_ref[...], w_ref[...], preferred_element_type=jnp.float32) + b_ref[...]


def _out_proj(h2, wout, bout2):
    M = B * L
    OD = wout.shape[1]
    return pl.pallas_call(
        _out_kernel,
        out_shape=jax.ShapeDtypeStruct((M, OD), jnp.float32),
        grid=(M // _MT,),
        in_specs=[
            pl.BlockSpec((_MT, D), lambda i: (i, 0)),
            pl.BlockSpec((D, OD), lambda i: (0, 0)),
            pl.BlockSpec((1, OD), lambda i: (0, 0)),
        ],
        out_specs=pl.BlockSpec((_MT, OD), lambda i: (i, 0)),
        compiler_params=pltpu.CompilerParams(
            dimension_semantics=("parallel",),
            vmem_limit_bytes=48 * 1024 * 1024,
        ),
        name="mamba_out",
        interpret=_INTERPRET,
    )(h2, wout, bout2)


def kernel(x, norm_w, Win, conv_w, conv_b, Wx, Wdt, bdt, A_log, Dp, Wblk,
           Wout, bout):
    h = x
    nw = norm_w[:, None, :]
    cwT = conv_w.transpose(0, 2, 1)
    cb = conv_b[:, None, :]
    bdt2 = bdt[:, None, :]
    a_logT = A_log.transpose(0, 2, 1)
    dp2 = Dp[:, None, :]
    pb = jnp.kron(jnp.eye(NS, dtype=jnp.float32),
                  jnp.ones((1, 128), jnp.float32))
    for l in range(Win.shape[0]):
        o = _proj(l, h, nw, Win, cwT, cb)
        delta, bmf, cmf = _dbc(l, o, Wx, Wdt, bdt2, pb)
        bm4 = bmf.reshape(B, L, NS, 128)
        cm4 = cmf.reshape(B, L, NS, 128)
        yz = _scan(l, o, delta, bm4, cm4, a_logT, dp2)
        h = _down(l, yz, Wblk, h)

    out2 = _out_proj(h.reshape(B * L, D), Wout, bout[None, :])
    return out2.reshape(B, L, Wout.shape[1])


# speedup vs baseline: 1.2453x; 1.0010x over previous
import jax
import jax.numpy as jnp
from jax.experimental import pallas as pl
from jax.experimental.pallas import tpu as pltpu

_INTERPRET = False

B, L, D = 2, 1024, 1024
DI = 2048
NS = 16
DTR = 64
KC = 4
EPS = 1e-5

TDI = 256
NB = DI // TDI
LT = 256
NLT = L // LT


def _silu(v):
    return v * jax.nn.sigmoid(v)


def _proj_kernel(x_ref, nw_ref, win_ref, cw_ref, cb_ref, o_ref, carry):
    j = pl.program_id(1)
    lt = pl.program_id(2)
    x = x_ref[0]
    ssq = jnp.mean(x * x, axis=-1, keepdims=True)
    xn = x * jax.lax.rsqrt(ssq + EPS) * nw_ref[0]
    xz = jnp.dot(xn, win_ref[0], preferred_element_type=jnp.float32)

    @pl.when(lt == 0)
    def _():
        carry[...] = jnp.zeros((KC - 1, DI), jnp.float32)

    w = cw_ref[0]
    xfull = jnp.concatenate([carry[...], xz], axis=0)
    carry[...] = xz[LT - (KC - 1):, :]
    acc = cb_ref[0]
    for k in range(KC):
        acc = acc + xfull[k:k + LT] * w[k][None, :]
    o_ref[0, 0] = jnp.where(j == 0, _silu(acc), xz)


def _proj(l, x, nw, win, cwT, cb):
    return pl.pallas_call(
        _proj_kernel,
        out_shape=jax.ShapeDtypeStruct((B, 2, L, DI), jnp.float32),
        grid=(B, 2, NLT),
        in_specs=[
            pl.BlockSpec((1, LT, D), lambda b, j, lt: (b, lt, 0)),
            pl.BlockSpec((1, 1, D), lambda b, j, lt: (l, 0, 0)),
            pl.BlockSpec((1, D, DI), lambda b, j, lt: (l, 0, j)),
            pl.BlockSpec((1, KC, DI), lambda b, j, lt: (l, 0, 0)),
            pl.BlockSpec((1, 1, DI), lambda b, j, lt: (l, 0, 0)),
        ],
        out_specs=pl.BlockSpec((1, 1, LT, DI), lambda b, j, lt: (b, j, lt, 0)),
        scratch_shapes=[pltpu.VMEM((KC - 1, DI), jnp.float32)],
        compiler_params=pltpu.CompilerParams(
            dimension_semantics=("parallel", "arbitrary", "arbitrary"),
            vmem_limit_bytes=48 * 1024 * 1024,
        ),
        name="mamba_proj",
        interpret=_INTERPRET,
    )(x, nw, win, cwT, cb)


_MT = 256
_NMT = B * L // _MT


def _dbc_kernel(xc_ref, wx_ref, wdt_ref, bdt_ref, pb_ref, d_ref, b_ref, c_ref):
    xc = xc_ref[0, 0]
    dbc = jnp.dot(xc, wx_ref[0], preferred_element_type=jnp.float32)
    dt = dbc[:, :DTR]
    d_ref[0] = jax.nn.softplus(
        jnp.dot(dt, wdt_ref[0], preferred_element_type=jnp.float32)
        + bdt_ref[0])
    pb = pb_ref[...]
    b_ref[0] = jnp.dot(dbc[:, DTR:DTR + NS], pb,
                       preferred_element_type=jnp.float32).astype(jnp.bfloat16)
    c_ref[0] = jnp.dot(dbc[:, DTR + NS:DTR + 2 * NS], pb,
                       preferred_element_type=jnp.float32).astype(jnp.bfloat16)


def _dbc(l, o, wx, wdt, bdt2, pb):
    nlt = L // _MT
    return pl.pallas_call(
        _dbc_kernel,
        out_shape=(
            jax.ShapeDtypeStruct((B, L, DI), jnp.float32),
            jax.ShapeDtypeStruct((B, L, NS * 128), jnp.bfloat16),
            jax.ShapeDtypeStruct((B, L, NS * 128), jnp.bfloat16),
        ),
        grid=(_NMT,),
        in_specs=[
            pl.BlockSpec((1, 1, _MT, DI), lambda i: (i // nlt, 0, i % nlt, 0)),
            pl.BlockSpec((1, DI, DTR + 2 * NS), lambda i: (l, 0, 0)),
            pl.BlockSpec((1, DTR, DI), lambda i: (l, 0, 0)),
            pl.BlockSpec((1, 1, DI), lambda i: (l, 0, 0)),
            pl.BlockSpec((NS, NS * 128), lambda i: (0, 0)),
        ],
        out_specs=(
            pl.BlockSpec((1, _MT, DI), lambda i: (i // nlt, i % nlt, 0)),
            pl.BlockSpec((1, _MT, NS * 128), lambda i: (i // nlt, i % nlt, 0)),
            pl.BlockSpec((1, _MT, NS * 128), lambda i: (i // nlt, i % nlt, 0)),
        ),
        compiler_params=pltpu.CompilerParams(
            dimension_semantics=("parallel",),
            vmem_limit_bytes=48 * 1024 * 1024,
        ),
        name="mamba_dbc",
        interpret=_INTERPRET,
    )(o, wx, wdt, bdt2, pb)


_PC = 128
L2 = L // 2


_RG = 8


def _scan_kernel(u_ref, d_ref, bm_ref, cm_ref, z_ref, at_ref, dp_ref,
                 o_ref):
    A = -jnp.exp(at_ref[0])
    rep = TDI // 128
    dp = dp_ref[0]

    def group(g, h):
        sl = pl.ds(g * _RG, _RG)
        d8 = d_ref[0, sl, :]
        u8 = u_ref[0, 0, sl, :]
        du8 = d8 * u8
        bm8 = bm_ref[0, sl, :, :].astype(jnp.float32)
        hs = []
        for i in range(_RG):
            dA = jnp.exp(jnp.broadcast_to(d8[i:i + 1, :], (NS, TDI)) * A)
            bmt = pltpu.repeat(bm8[i], rep, axis=1)
            dBu = jnp.broadcast_to(du8[i:i + 1, :], (NS, TDI)) * bmt
            h = dA * h + dBu
            hs.append(h)
        cm8 = cm_ref[0, sl, :, :].astype(jnp.float32)
        cmr = pltpu.repeat(cm8.reshape(_RG * NS, 128), rep,
                           axis=1).reshape(_RG, NS, TDI)
        y = jnp.sum(jnp.stack(hs, axis=0) * cmr, axis=1)
        y = y + u8 * dp
        z8 = z_ref[0, 0, sl, :]
        o_ref[0, sl, :] = y * _silu(z8)
        return h

    jax.lax.fori_loop(0, L // _RG, group,
                      jnp.zeros((NS, TDI), jnp.float32))


def _scan(l, o, delta, bm4, cm4, a_logT, dp2):
    grid = (B * NB,)
    return pl.pallas_call(
        _scan_kernel,
        out_shape=jax.ShapeDtypeStruct((B, L, DI), jnp.float32),
        grid=grid,
        in_specs=[
            pl.BlockSpec((1, 1, L, TDI), lambda i: (i // NB, 0, 0, i % NB)),
            pl.BlockSpec((1, L, TDI), lambda i: (i // NB, 0, i % NB)),
            pl.BlockSpec((1, L, NS, 128), lambda i: (i // NB, 0, 0, 0)),
            pl.BlockSpec((1, L, NS, 128), lambda i: (i // NB, 0, 0, 0)),
            pl.BlockSpec((1, 1, L, TDI), lambda i: (i // NB, 1, 0, i % NB)),
            pl.BlockSpec((1, NS, TDI), lambda i: (l, 0, i % NB)),
            pl.BlockSpec((1, 1, TDI), lambda i: (l, 0, i % NB)),
        ],
        out_specs=pl.BlockSpec((1, L, TDI), lambda i: (i // NB, 0, i % NB)),

        compiler_params=pltpu.CompilerParams(
            dimension_semantics=("parallel",),
            vmem_limit_bytes=55 * 1024 * 1024,
        ),
        name="mamba_scan",
        interpret=_INTERPRET,
    )(o, delta, bm4, cm4, o, a_logT, dp2)


def _down_kernel(yz_ref, w_ref, x_ref, o_ref):
    o_ref[0] = x_ref[0] + jnp.dot(
        yz_ref[0], w_ref[0], preferred_element_type=jnp.float32)


def _down(l, yz, wblk, x):
    nlt = L // _MT
    return pl.pallas_call(
        _down_kernel,
        out_shape=jax.ShapeDtypeStruct((B, L, D), jnp.float32),
        grid=(_NMT,),
        in_specs=[
            pl.BlockSpec((1, _MT, DI), lambda i: (i // nlt, i % nlt, 0)),
            pl.BlockSpec((1, DI, D), lambda i: (l, 0, 0)),
            pl.BlockSpec((1, _MT, D), lambda i: (i // nlt, i % nlt, 0)),
        ],
        out_specs=pl.BlockSpec((1, _MT, D), lambda i: (i // nlt, i % nlt, 0)),
        compiler_params=pltpu.CompilerParams(
            dimension_semantics=("parallel",),
            vmem_limit_bytes=48 * 1024 * 1024,
        ),
        name="mamba_down",
        interpret=_INTERPRET,
    )(yz, wblk, x)


def _out_kernel(h_ref, w_ref, b_ref, o_ref):
    o_ref[...] = jnp.dot(
        h_ref[...], w_ref[...], preferred_element_type=jnp.float32) + b_ref[...]


def _out_proj(h2, wout, bout2):
    M = B * L
    OD = wout.shape[1]
    return pl.pallas_call(
        _out_kernel,
        out_shape=jax.ShapeDtypeStruct((M, OD), jnp.float32),
        grid=(M // _MT,),
        in_specs=[
            pl.BlockSpec((_MT, D), lambda i: (i, 0)),
            pl.BlockSpec((D, OD), lambda i: (0, 0)),
            pl.BlockSpec((1, OD), lambda i: (0, 0)),
        ],
        out_specs=pl.BlockSpec((_MT, OD), lambda i: (i, 0)),
        compiler_params=pltpu.CompilerParams(
            dimension_semantics=("parallel",),
            vmem_limit_bytes=48 * 1024 * 1024,
        ),
        name="mamba_out",
        interpret=_INTERPRET,
    )(h2, wout, bout2)


def kernel(x, norm_w, Win, conv_w, conv_b, Wx, Wdt, bdt, A_log, Dp, Wblk,
           Wout, bout):
    h = x
    nw = norm_w[:, None, :]
    cwT = conv_w.transpose(0, 2, 1)
    cb = conv_b[:, None, :]
    bdt2 = bdt[:, None, :]
    a_logT = A_log.transpose(0, 2, 1)
    dp2 = Dp[:, None, :]
    pb = jnp.kron(jnp.eye(NS, dtype=jnp.float32),
                  jnp.ones((1, 128), jnp.float32))
    for l in range(Win.shape[0]):
        o = _proj(l, h, nw, Win, cwT, cb)
        delta, bmf, cmf = _dbc(l, o, Wx, Wdt, bdt2, pb)
        bm4 = bmf.reshape(B, L, NS, 128)
        cm4 = cmf.reshape(B, L, NS, 128)
        yz = _scan(l, o, delta, bm4, cm4, a_logT, dp2)
        h = _down(l, yz, Wblk, h)

    out2 = _out_proj(h.reshape(B * L, D), Wout, bout[None, :])
    return out2.reshape(B, L, Wout.shape[1])


# speedup vs baseline: 1.2471x; 1.0015x over previous
import jax
import jax.numpy as jnp
from jax.experimental import pallas as pl
from jax.experimental.pallas import tpu as pltpu

_INTERPRET = False

B, L, D = 2, 1024, 1024
DI = 2048
NS = 16
DTR = 64
KC = 4
EPS = 1e-5

TDI = 256
NB = DI // TDI
LT = 256
NLT = L // LT


def _silu(v):
    return v * jax.nn.sigmoid(v)


def _proj_kernel(x_ref, nw_ref, win_ref, cw_ref, cb_ref, o_ref, carry):
    j = pl.program_id(1)
    lt = pl.program_id(2)
    x = x_ref[0]
    ssq = jnp.mean(x * x, axis=-1, keepdims=True)
    xn = x * jax.lax.rsqrt(ssq + EPS) * nw_ref[0]
    xz = jnp.dot(xn, win_ref[0], preferred_element_type=jnp.float32)

    @pl.when(lt == 0)
    def _():
        carry[...] = jnp.zeros((KC - 1, DI), jnp.float32)

    w = cw_ref[0]
    xfull = jnp.concatenate([carry[...], xz], axis=0)
    carry[...] = xz[LT - (KC - 1):, :]
    acc = cb_ref[0]
    for k in range(KC):
        acc = acc + xfull[k:k + LT] * w[k][None, :]
    o_ref[0, 0] = jnp.where(j == 0, _silu(acc), xz).astype(jnp.bfloat16)


def _proj(l, x, nw, win, cwT, cb):
    return pl.pallas_call(
        _proj_kernel,
        out_shape=jax.ShapeDtypeStruct((B, 2, L, DI), jnp.bfloat16),
        grid=(B, 2, NLT),
        in_specs=[
            pl.BlockSpec((1, LT, D), lambda b, j, lt: (b, lt, 0)),
            pl.BlockSpec((1, 1, D), lambda b, j, lt: (l, 0, 0)),
            pl.BlockSpec((1, D, DI), lambda b, j, lt: (l, 0, j)),
            pl.BlockSpec((1, KC, DI), lambda b, j, lt: (l, 0, 0)),
            pl.BlockSpec((1, 1, DI), lambda b, j, lt: (l, 0, 0)),
        ],
        out_specs=pl.BlockSpec((1, 1, LT, DI), lambda b, j, lt: (b, j, lt, 0)),
        scratch_shapes=[pltpu.VMEM((KC - 1, DI), jnp.float32)],
        compiler_params=pltpu.CompilerParams(
            dimension_semantics=("parallel", "arbitrary", "arbitrary"),
            vmem_limit_bytes=48 * 1024 * 1024,
        ),
        name="mamba_proj",
        interpret=_INTERPRET,
    )(x, nw, win, cwT, cb)


_MT = 256
_NMT = B * L // _MT


def _dbc_kernel(xc_ref, wx_ref, wdt_ref, bdt_ref, pb_ref, d_ref, b_ref, c_ref):
    xc = xc_ref[0, 0].astype(jnp.float32)
    dbc = jnp.dot(xc, wx_ref[0], preferred_element_type=jnp.float32)
    dt = dbc[:, :DTR]
    d_ref[0] = jax.nn.softplus(
        jnp.dot(dt, wdt_ref[0], preferred_element_type=jnp.float32)
        + bdt_ref[0])
    pb = pb_ref[...]
    b_ref[0] = jnp.dot(dbc[:, DTR:DTR + NS], pb,
                       preferred_element_type=jnp.float32).astype(jnp.bfloat16)
    c_ref[0] = jnp.dot(dbc[:, DTR + NS:DTR + 2 * NS], pb,
                       preferred_element_type=jnp.float32).astype(jnp.bfloat16)


def _dbc(l, o, wx, wdt, bdt2, pb):
    nlt = L // _MT
    return pl.pallas_call(
        _dbc_kernel,
        out_shape=(
            jax.ShapeDtypeStruct((B, L, DI), jnp.float32),
            jax.ShapeDtypeStruct((B, L, NS * 128), jnp.bfloat16),
            jax.ShapeDtypeStruct((B, L, NS * 128), jnp.bfloat16),
        ),
        grid=(_NMT,),
        in_specs=[
            pl.BlockSpec((1, 1, _MT, DI), lambda i: (i // nlt, 0, i % nlt, 0)),
            pl.BlockSpec((1, DI, DTR + 2 * NS), lambda i: (l, 0, 0)),
            pl.BlockSpec((1, DTR, DI), lambda i: (l, 0, 0)),
            pl.BlockSpec((1, 1, DI), lambda i: (l, 0, 0)),
            pl.BlockSpec((NS, NS * 128), lambda i: (0, 0)),
        ],
        out_specs=(
            pl.BlockSpec((1, _MT, DI), lambda i: (i // nlt, i % nlt, 0)),
            pl.BlockSpec((1, _MT, NS * 128), lambda i: (i // nlt, i % nlt, 0)),
            pl.BlockSpec((1, _MT, NS * 128), lambda i: (i // nlt, i % nlt, 0)),
        ),
        compiler_params=pltpu.CompilerParams(
            dimension_semantics=("parallel",),
            vmem_limit_bytes=48 * 1024 * 1024,
        ),
        name="mamba_dbc",
        interpret=_INTERPRET,
    )(o, wx, wdt, bdt2, pb)


_PC = 128
L2 = L // 2


_RG = 8


def _scan_kernel(u_ref, d_ref, bm_ref, cm_ref, z_ref, at_ref, dp_ref,
                 o_ref):
    A = -jnp.exp(at_ref[0])
    rep = TDI // 128
    dp = dp_ref[0]

    def group(g, h):
        sl = pl.ds(g * _RG, _RG)
        d8 = d_ref[0, sl, :]
        u8 = u_ref[0, 0, sl, :].astype(jnp.float32)
        du8 = d8 * u8
        bm8 = bm_ref[0, sl, :, :].astype(jnp.float32)
        hs = []
        for i in range(_RG):
            dA = jnp.exp(jnp.broadcast_to(d8[i:i + 1, :], (NS, TDI)) * A)
            bmt = pltpu.repeat(bm8[i], rep, axis=1)
            dBu = jnp.broadcast_to(du8[i:i + 1, :], (NS, TDI)) * bmt
            h = dA * h + dBu
            hs.append(h)
        cm8 = cm_ref[0, sl, :, :].astype(jnp.float32)
        cmr = pltpu.repeat(cm8.reshape(_RG * NS, 128), rep,
                           axis=1).reshape(_RG, NS, TDI)
        y = jnp.sum(jnp.stack(hs, axis=0) * cmr, axis=1)
        y = y + u8 * dp
        z8 = z_ref[0, 0, sl, :].astype(jnp.float32)
        o_ref[0, sl, :] = y * _silu(z8)
        return h

    jax.lax.fori_loop(0, L // _RG, group,
                      jnp.zeros((NS, TDI), jnp.float32))


def _scan(l, o, delta, bm4, cm4, a_logT, dp2):
    grid = (B * NB,)
    return pl.pallas_call(
        _scan_kernel,
        out_shape=jax.ShapeDtypeStruct((B, L, DI), jnp.float32),
        grid=grid,
        in_specs=[
            pl.BlockSpec((1, 1, L, TDI), lambda i: (i // NB, 0, 0, i % NB)),
            pl.BlockSpec((1, L, TDI), lambda i: (i // NB, 0, i % NB)),
            pl.BlockSpec((1, L, NS, 128), lambda i: (i // NB, 0, 0, 0)),
            pl.BlockSpec((1, L, NS, 128), lambda i: (i // NB, 0, 0, 0)),
            pl.BlockSpec((1, 1, L, TDI), lambda i: (i // NB, 1, 0, i % NB)),
            pl.BlockSpec((1, NS, TDI), lambda i: (l, 0, i % NB)),
            pl.BlockSpec((1, 1, TDI), lambda i: (l, 0, i % NB)),
        ],
        out_specs=pl.BlockSpec((1, L, TDI), lambda i: (i // NB, 0, i % NB)),

        compiler_params=pltpu.CompilerParams(
            dimension_semantics=("parallel",),
            vmem_limit_bytes=55 * 1024 * 1024,
        ),
        name="mamba_scan",
        interpret=_INTERPRET,
    )(o, delta, bm4, cm4, o, a_logT, dp2)


def _down_kernel(yz_ref, w_ref, x_ref, o_ref):
    o_ref[0] = x_ref[0] + jnp.dot(
        yz_ref[0], w_ref[0], preferred_element_type=jnp.float32)


def _down(l, yz, wblk, x):
    nlt = L // _MT
    return pl.pallas_call(
        _down_kernel,
        out_shape=jax.ShapeDtypeStruct((B, L, D), jnp.float32),
        grid=(_NMT,),
        in_specs=[
            pl.BlockSpec((1, _MT, DI), lambda i: (i // nlt, i % nlt, 0)),
            pl.BlockSpec((1, DI, D), lambda i: (l, 0, 0)),
            pl.BlockSpec((1, _MT, D), lambda i: (i // nlt, i % nlt, 0)),
        ],
        out_specs=pl.BlockSpec((1, _MT, D), lambda i: (i // nlt, i % nlt, 0)),
        compiler_params=pltpu.CompilerParams(
            dimension_semantics=("parallel",),
            vmem_limit_bytes=48 * 1024 * 1024,
        ),
        name="mamba_down",
        interpret=_INTERPRET,
    )(yz, wblk, x)


def _out_kernel(h_ref, w_ref, b_ref, o_ref):
    o_ref[...] = jnp.dot(
        h_ref[...], w_ref[...], preferred_element_type=jnp.float32) + b_ref[...]


def _out_proj(h2, wout, bout2):
    M = B * L
    OD = wout.shape[1]
    return pl.pallas_call(
        _out_kernel,
        out_shape=jax.ShapeDtypeStruct((M, OD), jnp.float32),
        grid=(M // _MT,),
        in_specs=[
            pl.BlockSpec((_MT, D), lambda i: (i, 0)),
            pl.BlockSpec((D, OD), lambda i: (0, 0)),
            pl.BlockSpec((1, OD), lambda i: (0, 0)),
        ],
        out_specs=pl.BlockSpec((_MT, OD), lambda i: (i, 0)),
        compiler_params=pltpu.CompilerParams(
            dimension_semantics=("parallel",),
            vmem_limit_bytes=48 * 1024 * 1024,
        ),
        name="mamba_out",
        interpret=_INTERPRET,
    )(h2, wout, bout2)


def kernel(x, norm_w, Win, conv_w, conv_b, Wx, Wdt, bdt, A_log, Dp, Wblk,
           Wout, bout):
    h = x
    nw = norm_w[:, None, :]
    cwT = conv_w.transpose(0, 2, 1)
    cb = conv_b[:, None, :]
    bdt2 = bdt[:, None, :]
    a_logT = A_log.transpose(0, 2, 1)
    dp2 = Dp[:, None, :]
    pb = jnp.kron(jnp.eye(NS, dtype=jnp.float32),
                  jnp.ones((1, 128), jnp.float32))
    for l in range(Win.shape[0]):
        o = _proj(l, h, nw, Win, cwT, cb)
        delta, bmf, cmf = _dbc(l, o, Wx, Wdt, bdt2, pb)
        bm4 = bmf.reshape(B, L, NS, 128)
        cm4 = cmf.reshape(B, L, NS, 128)
        yz = _scan(l, o, delta, bm4, cm4, a_logT, dp2)
        h = _down(l, yz, Wblk, h)

    out2 = _out_proj(h.reshape(B * L, D), Wout, bout[None, :])
    return out2.reshape(B, L, Wout.shape[1])


# speedup vs baseline: 1.2483x; 1.0009x over previous
import jax
import jax.numpy as jnp
from jax.experimental import pallas as pl
from jax.experimental.pallas import tpu as pltpu

B, L, D = 2, 1024, 1024
DI = 2048
NS = 16
DTR = 64
KC = 4
EPS = 1e-5

TDI = 256
NB = DI // TDI
LT = 256
NLT = L // LT


def _silu(v):
    return v * jax.nn.sigmoid(v)


def _proj_kernel(x_ref, nw_ref, win_ref, cw_ref, cb_ref, o_ref, carry):
    j = pl.program_id(1)
    lt = pl.program_id(2)
    x = x_ref[0]
    ssq = jnp.mean(x * x, axis=-1, keepdims=True)
    xn = x * jax.lax.rsqrt(ssq + EPS) * nw_ref[0]
    xz = jnp.dot(xn, win_ref[0], preferred_element_type=jnp.float32)

    @pl.when(lt == 0)
    def _():
        carry[...] = jnp.zeros((KC - 1, DI), jnp.float32)

    w = cw_ref[0]
    xfull = jnp.concatenate([carry[...], xz], axis=0)
    carry[...] = xz[LT - (KC - 1):, :]
    acc = cb_ref[0]
    for k in range(KC):
        acc = acc + xfull[k:k + LT] * w[k][None, :]
    o_ref[0, 0] = jnp.where(j == 0, _silu(acc), xz).astype(jnp.bfloat16)


def _proj(l, x, nw, win, cwT, cb):
    return pl.pallas_call(
        _proj_kernel,
        out_shape=jax.ShapeDtypeStruct((B, 2, L, DI), jnp.bfloat16),
        grid=(B, 2, NLT),
        in_specs=[
            pl.BlockSpec((1, LT, D), lambda b, j, lt: (b, lt, 0)),
            pl.BlockSpec((1, 1, D), lambda b, j, lt: (l, 0, 0)),
            pl.BlockSpec((1, D, DI), lambda b, j, lt: (l, 0, j)),
            pl.BlockSpec((1, KC, DI), lambda b, j, lt: (l, 0, 0)),
            pl.BlockSpec((1, 1, DI), lambda b, j, lt: (l, 0, 0)),
        ],
        out_specs=pl.BlockSpec((1, 1, LT, DI), lambda b, j, lt: (b, j, lt, 0)),
        scratch_shapes=[pltpu.VMEM((KC - 1, DI), jnp.float32)],
        compiler_params=pltpu.CompilerParams(
            dimension_semantics=("parallel", "arbitrary", "arbitrary"),
            vmem_limit_bytes=48 * 1024 * 1024,
        ),
        name="mamba_proj",
    )(x, nw, win, cwT, cb)


_MT = 256
_NMT = B * L // _MT


def _dbc_kernel(xc_ref, wx_ref, wdt_ref, bdt_ref, pb_ref, d_ref, b_ref, c_ref):
    xc = xc_ref[0, 0].astype(jnp.float32)
    dbc = jnp.dot(xc, wx_ref[0], preferred_element_type=jnp.float32)
    dt = dbc[:, :DTR]
    d_ref[0] = jax.nn.softplus(
        jnp.dot(dt, wdt_ref[0], preferred_element_type=jnp.float32)
        + bdt_ref[0])
    pb = pb_ref[...]
    b_ref[0] = jnp.dot(dbc[:, DTR:DTR + NS], pb,
                       preferred_element_type=jnp.float32).astype(jnp.bfloat16)
    c_ref[0] = jnp.dot(dbc[:, DTR + NS:DTR + 2 * NS], pb,
                       preferred_element_type=jnp.float32).astype(jnp.bfloat16)


def _dbc(l, o, wx, wdt, bdt2, pb):
    nlt = L // _MT
    return pl.pallas_call(
        _dbc_kernel,
        out_shape=(
            jax.ShapeDtypeStruct((B, L, DI), jnp.float32),
            jax.ShapeDtypeStruct((B, L, NS * 128), jnp.bfloat16),
            jax.ShapeDtypeStruct((B, L, NS * 128), jnp.bfloat16),
        ),
        grid=(_NMT,),
        in_specs=[
            pl.BlockSpec((1, 1, _MT, DI), lambda i: (i // nlt, 0, i % nlt, 0)),
            pl.BlockSpec((1, DI, DTR + 2 * NS), lambda i: (l, 0, 0)),
            pl.BlockSpec((1, DTR, DI), lambda i: (l, 0, 0)),
            pl.BlockSpec((1, 1, DI), lambda i: (l, 0, 0)),
            pl.BlockSpec((NS, NS * 128), lambda i: (0, 0)),
        ],
        out_specs=(
            pl.BlockSpec((1, _MT, DI), lambda i: (i // nlt, i % nlt, 0)),
            pl.BlockSpec((1, _MT, NS * 128), lambda i: (i // nlt, i % nlt, 0)),
            pl.BlockSpec((1, _MT, NS * 128), lambda i: (i // nlt, i % nlt, 0)),
        ),
        compiler_params=pltpu.CompilerParams(
            dimension_semantics=("parallel",),
            vmem_limit_bytes=48 * 1024 * 1024,
        ),
        name="mamba_dbc",
    )(o, wx, wdt, bdt2, pb)


_PC = 128
L2 = L // 2


_RG = 8


def _scan_kernel(u_ref, d_ref, bm_ref, cm_ref, z_ref, at_ref, dp_ref,
                 o_ref):
    A = -jnp.exp(at_ref[0])
    rep = TDI // 128
    dp = dp_ref[0]

    def group(g, h):
        sl = pl.ds(g * _RG, _RG)
        d8 = d_ref[0, sl, :]
        u8 = u_ref[0, 0, sl, :].astype(jnp.float32)
        du8 = d8 * u8
        bm8 = bm_ref[0, sl, :, :].astype(jnp.float32)
        hs = []
        for i in range(_RG):
            dA = jnp.exp(jnp.broadcast_to(d8[i:i + 1, :], (NS, TDI)) * A)
            bmt = pltpu.repeat(bm8[i], rep, axis=1)
            dBu = jnp.broadcast_to(du8[i:i + 1, :], (NS, TDI)) * bmt
            h = dA * h + dBu
            hs.append(h)
        cm8 = cm_ref[0, sl, :, :].astype(jnp.float32)
        cmr = pltpu.repeat(cm8.reshape(_RG * NS, 128), rep,
                           axis=1).reshape(_RG, NS, TDI)
        y = jnp.sum(jnp.stack(hs, axis=0) * cmr, axis=1)
        y = y + u8 * dp
        z8 = z_ref[0, 0, sl, :].astype(jnp.float32)
        o_ref[0, sl, :] = y * _silu(z8)
        return h

    jax.lax.fori_loop(0, L // _RG, group,
                      jnp.zeros((NS, TDI), jnp.float32))


def _scan(l, o, delta, bm4, cm4, a_logT, dp2):
    grid = (B * NB,)
    return pl.pallas_call(
        _scan_kernel,
        out_shape=jax.ShapeDtypeStruct((B, L, DI), jnp.float32),
        grid=grid,
        in_specs=[
            pl.BlockSpec((1, 1, L, TDI), lambda i: (i // NB, 0, 0, i % NB)),
            pl.BlockSpec((1, L, TDI), lambda i: (i // NB, 0, i % NB)),
            pl.BlockSpec((1, L, NS, 128), lambda i: (i // NB, 0, 0, 0)),
            pl.BlockSpec((1, L, NS, 128), lambda i: (i // NB, 0, 0, 0)),
            pl.BlockSpec((1, 1, L, TDI), lambda i: (i // NB, 1, 0, i % NB)),
            pl.BlockSpec((1, NS, TDI), lambda i: (l, 0, i % NB)),
            pl.BlockSpec((1, 1, TDI), lambda i: (l, 0, i % NB)),
        ],
        out_specs=pl.BlockSpec((1, L, TDI), lambda i: (i // NB, 0, i % NB)),

        compiler_params=pltpu.CompilerParams(
            dimension_semantics=("parallel",),
            vmem_limit_bytes=55 * 1024 * 1024,
        ),
        name="mamba_scan",
    )(o, delta, bm4, cm4, o, a_logT, dp2)


def _down_kernel(yz_ref, w_ref, x_ref, o_ref):
    o_ref[0] = x_ref[0] + jnp.dot(
        yz_ref[0], w_ref[0], preferred_element_type=jnp.float32)


def _down(l, yz, wblk, x):
    nlt = L // _MT
    return pl.pallas_call(
        _down_kernel,
        out_shape=jax.ShapeDtypeStruct((B, L, D), jnp.float32),
        grid=(_NMT,),
        in_specs=[
            pl.BlockSpec((1, _MT, DI), lambda i: (i // nlt, i % nlt, 0)),
            pl.BlockSpec((1, DI, D), lambda i: (l, 0, 0)),
            pl.BlockSpec((1, _MT, D), lambda i: (i // nlt, i % nlt, 0)),
        ],
        out_specs=pl.BlockSpec((1, _MT, D), lambda i: (i // nlt, i % nlt, 0)),
        compiler_params=pltpu.CompilerParams(
            dimension_semantics=("parallel",),
            vmem_limit_bytes=48 * 1024 * 1024,
        ),
        name="mamba_down",
    )(yz, wblk, x)


def _out_kernel(h_ref, w_ref, b_ref, o_ref):
    o_ref[...] = jnp.dot(
        h_ref[...], w_ref[...], preferred_element_type=jnp.float32) + b_ref[...]


def _out_proj(h2, wout, bout2):
    M = B * L
    OD = wout.shape[1]
    return pl.pallas_call(
        _out_kernel,
        out_shape=jax.ShapeDtypeStruct((M, OD), jnp.float32),
        grid=(M // _MT,),
        in_specs=[
            pl.BlockSpec((_MT, D), lambda i: (i, 0)),
            pl.BlockSpec((D, OD), lambda i: (0, 0)),
            pl.BlockSpec((1, OD), lambda i: (0, 0)),
        ],
        out_specs=pl.BlockSpec((_MT, OD), lambda i: (i, 0)),
        compiler_params=pltpu.CompilerParams(
            dimension_semantics=("parallel",),
            vmem_limit_bytes=48 * 1024 * 1024,
        ),
        name="mamba_out",
    )(h2, wout, bout2)


def kernel(x, norm_w, Win, conv_w, conv_b, Wx, Wdt, bdt, A_log, Dp, Wblk,
           Wout, bout):
    h = x
    nw = norm_w[:, None, :]
    cwT = conv_w.transpose(0, 2, 1)
    cb = conv_b[:, None, :]
    bdt2 = bdt[:, None, :]
    a_logT = A_log.transpose(0, 2, 1)
    dp2 = Dp[:, None, :]
    pb = jnp.kron(jnp.eye(NS, dtype=jnp.float32),
                  jnp.ones((1, 128), jnp.float32))
    for l in range(Win.shape[0]):
        o = _proj(l, h, nw, Win, cwT, cb)
        delta, bmf, cmf = _dbc(l, o, Wx, Wdt, bdt2, pb)
        bm4 = bmf.reshape(B, L, NS, 128)
        cm4 = cmf.reshape(B, L, NS, 128)
        yz = _scan(l, o, delta, bm4, cm4, a_logT, dp2)
        h = _down(l, yz, Wblk, h)

    out2 = _out_proj(h.reshape(B * L, D), Wout, bout[None, :])
    return out2.reshape(B, L, Wout.shape[1])
